# baseline (speedup 1.0000x reference)
import jax
import jax.numpy as jnp
from jax import lax
from jax.experimental import pallas as pl
from jax.experimental.pallas import tpu as pltpu
from jax.experimental.pallas import tpu_sc as plsc

_EPS = 1e-5
_NCORES = 2
_NSUB = 16
_K = 128


def _make_agg(n, e, fh, with_counts):
    ncs = e // (_K * _NSUB)
    nquad = ncs // 4
    na = n + 8
    nr = (na // _NSUB) // 8 * 8
    rem = na - nr * _NSUB
    mesh = plsc.VectorSubcoreMesh(core_axis_name="c", subcore_axis_name="s")

    nbuf = 4
    cw = 8
    out_type = [jax.ShapeDtypeStruct((_NCORES, na, fh), jnp.float32)]
    scratch = [
        pltpu.VMEM((ncs, _K), jnp.int32),
        pltpu.VMEM((ncs, _K), jnp.int32),
        pltpu.VMEM_SHARED((na, fh), jnp.float32),
    ] + [pltpu.VMEM((_K, fh), jnp.float32) for _ in range(nbuf)] \
      + [pltpu.SemaphoreType.DMA for _ in range(2 * nbuf)]
    if with_counts:
        out_type.append(jax.ShapeDtypeStruct((na, cw), jnp.float32))
        scratch += [
            pltpu.VMEM((_K, cw), jnp.float32),
            pltpu.VMEM((_K, cw), jnp.float32),
            pltpu.VMEM_SHARED((na, cw), jnp.float32),
            pltpu.SemaphoreType.DMA,
        ]

    def body(p_hbm, src_hbm, dst_hbm, *rest):
        if with_counts:
            (ones_hbm, zer_hbm, out_hbm, cnt_hbm, srcb, dstb, acc_sh,
             *tl) = rest
            rows = tl[:nbuf]
            gsem = tl[nbuf:2 * nbuf]
            ssem = tl[2 * nbuf:3 * nbuf]
            ones, zb, cnt_sh, csem = tl[3 * nbuf:]
        else:
            (out_hbm, srcb, dstb, acc_sh, *tl) = rest
            rows = tl[:nbuf]
            gsem = tl[nbuf:2 * nbuf]
            ssem = tl[2 * nbuf:3 * nbuf]
        c = lax.axis_index("c")
        s = lax.axis_index("s")

        zeros16 = jnp.zeros((16,), jnp.float32)

        def zrow(r, _):
            for j in range(fh // 16):
                rows[0][r, pl.ds(j * 16, 16)] = zeros16
            return 0

        lax.fori_loop(0, _K, zrow, 0)

        base = s * nr
        full, tail = nr // _K, nr % _K

        def zfill(dst_sh, zsrc):
            for j in range(full):
                pltpu.sync_copy(zsrc.at[pl.ds(0, _K)],
                                dst_sh.at[pl.ds(base + j * _K, _K)])
            if tail:
                pltpu.sync_copy(zsrc.at[pl.ds(0, tail)],
                                dst_sh.at[pl.ds(base + full * _K, tail)])
            if rem:
                @pl.when(s == _NSUB - 1)
                def _():
                    pltpu.sync_copy(zsrc.at[pl.ds(0, rem)],
                                    dst_sh.at[pl.ds(nr * _NSUB, rem)])

        zfill(acc_sh, rows[0])

        if with_counts:
            pltpu.sync_copy(ones_hbm, ones)
            pltpu.sync_copy(zer_hbm, zb)

            @pl.when(c == 0)
            def _():
                zfill(cnt_sh, zb)

        plsc.subcore_barrier()

        row_off = c * n
        pltpu.sync_copy(src_hbm.at[pl.ds(s * ncs, ncs)], srcb)
        pltpu.sync_copy(dst_hbm.at[pl.ds(s * ncs, ncs)], dstb)

        def fixrow(r, _):
            for j in range(_K // 16):
                sl = pl.ds(j * 16, 16)
                srcb[r, sl] = srcb[r, sl] + row_off
            return 0

        lax.fori_loop(0, ncs, fixrow, 0)

        def gstart(j, b):
            pltpu.async_copy(p_hbm.at[srcb.at[j]], rows[b], gsem[b])

        def gwait(j, b):
            pltpu.make_async_copy(p_hbm.at[srcb.at[j]], rows[b],
                                  gsem[b]).wait()

        def sstart(j, b):
            pltpu.async_copy(rows[b], acc_sh.at[dstb.at[j]], ssem[b],
                             add=True)
            if with_counts:
                @pl.when(c == 0)
                def _():
                    pltpu.async_copy(ones, cnt_sh.at[dstb.at[j]], csem,
                                     add=True)

        def swait(j, b):
            pltpu.make_async_copy(rows[b], acc_sh.at[dstb.at[j]],
                                  ssem[b]).wait()

        for b in range(nbuf):
            gstart(b, b)

        def quad(t, _):
            j0 = nbuf * t
            for b in range(nbuf):
                gwait(j0 + b, b)
                sstart(j0 + b, b)
            for b in range(nbuf):
                jn = j0 + b + nbuf

                @pl.when(jn < ncs)
                def _(b=b, j=j0 + b, jn=jn):
                    swait(j, b)
                    gstart(jn, b)

            return 0

        lax.fori_loop(0, nquad, quad, 0)

        for j in range(nquad * nbuf, ncs):
            gwait(j, j % nbuf)
            sstart(j, j % nbuf)
        for j in range(ncs - nbuf, ncs):
            swait(j, j % nbuf)
        if with_counts:
            @pl.when(c == 0)
            def _():
                def cdrain(j, _):
                    pltpu.make_async_copy(ones, cnt_sh.at[dstb.at[0]],
                                          csem).wait()
                    return 0

                lax.fori_loop(0, ncs, cdrain, 0)

        plsc.subcore_barrier()

        def wb(src_sh, dst_hbm_full, lead):
            dst3 = dst_hbm_full.at[lead] if lead is not None else dst_hbm_full
            pltpu.sync_copy(src_sh.at[pl.ds(base, nr)],
                            dst3.at[pl.ds(base, nr)])
            if rem:
                @pl.when(s == _NSUB - 1)
                def _():
                    pltpu.sync_copy(src_sh.at[pl.ds(nr * _NSUB, rem)],
                                    dst3.at[pl.ds(nr * _NSUB, rem)])

        wb(acc_sh, out_hbm, c)
        if with_counts:
            @pl.when(c == 0)
            def _():
                wb(cnt_sh, cnt_hbm, None)

    k = pl.kernel(body, out_type=out_type, mesh=mesh, scratch_types=scratch,
                  compiler_params=pltpu.CompilerParams(
                      use_tc_tiling_on_sc=False))
    if with_counts:
        return k
    return lambda *a: k(*a)[0]


_DN = (((1,), (1,)), ((), ()))


def _split_store(pp, p_ref):
    n, f2 = pp.shape
    fh = f2 // 2
    p_ref[:n, :] = pp[:, :fh]
    p_ref[n:, :] = pp[:, fh:]


def _s0_body(x_ref, wl_ref, wr_ref, b_ref, p_ref, r_ref):
    x = x_ref[...]
    pp = lax.dot_general(x, wl_ref[...], _DN,
                         preferred_element_type=jnp.float32)
    _split_store(pp, p_ref)
    r_ref[...] = lax.dot_general(x, wr_ref[...], _DN,
                                 preferred_element_type=jnp.float32) + b_ref[...]


def _smid_body(a_ref, c_ref, r_ref, g_ref, be_ref, wl_ref, wr_ref, b_ref,
               p_ref, rn_ref):
    n = r_ref.shape[0]
    cnt = jnp.maximum(c_ref[:n, :1], 1.0)
    a = a_ref[...]
    agg = jnp.concatenate([a[0, :n], a[1, :n]], axis=1)
    z = agg / cnt + r_ref[...]
    mu = jnp.mean(z, axis=0, keepdims=True)
    zc = z - mu
    var = jnp.mean(zc * zc, axis=0, keepdims=True)
    h = zc * lax.rsqrt(var + _EPS) * g_ref[...] + be_ref[...]
    h = jnp.maximum(h, 0.0)
    pp = lax.dot_general(h, wl_ref[...], _DN,
                         preferred_element_type=jnp.float32)
    _split_store(pp, p_ref)
    rn_ref[...] = lax.dot_general(h, wr_ref[...], _DN,
                                  preferred_element_type=jnp.float32) + b_ref[...]


def _sfin_body(a_ref, c_ref, r_ref, o_ref):
    n = r_ref.shape[0]
    cnt = jnp.maximum(c_ref[:n, :1], 1.0)
    a = a_ref[...]
    agg = jnp.concatenate([a[0, :n], a[1, :n]], axis=1)
    o_ref[...] = agg / cnt + r_ref[...]


def _sds(shape):
    return jax.ShapeDtypeStruct(shape, jnp.float32)


def kernel(x, edge_index, W1l, b1, W1r, g1, be1, W2l, b2, W2r, g2, be2,
           W3l, b3, W3r):
    n, d_in = x.shape
    e = edge_index.shape[1]
    d_hid = W1l.shape[0]
    d_out = W3l.shape[0]
    f3 = 64
    fh = d_hid // 2
    fh3 = f3 // 2

    grp = _K * _NSUB * 4
    e_pad = -(-e // grp) * grp
    src = jnp.pad(edge_index[0], (0, e_pad - e)).reshape(e_pad // _K, _K)
    dst = jnp.pad(edge_index[1], (0, e_pad - e),
                  constant_values=n).reshape(e_pad // _K, _K)
    ones_c = jnp.ones((_K, 8), jnp.float32)
    zer_c = jnp.zeros((_K, 8), jnp.float32)

    pad = f3 - d_out
    W3lp = jnp.pad(W3l, ((0, pad), (0, 0)))
    W3rp = jnp.pad(W3r, ((0, pad), (0, 0)))
    b3p = jnp.pad(b3, (0, pad))

    agg1 = _make_agg(n, e_pad, fh, True)
    agg2 = _make_agg(n, e_pad, fh, False)
    agg3 = _make_agg(n, e_pad, fh3, False)

    p1, r1 = pl.pallas_call(
        _s0_body,
        out_shape=[_sds((2 * n, fh)), _sds((n, d_hid))])(
            x, W1l, W1r, b1[None, :])

    a1, cnt = agg1(p1, src, dst, ones_c, zer_c)

    p2, r2 = pl.pallas_call(
        _smid_body,
        out_shape=[_sds((2 * n, fh)), _sds((n, d_hid))])(
            a1, cnt, r1, g1[None, :], be1[None, :], W2l, W2r, b2[None, :])

    a2 = agg2(p2, src, dst)

    p3, r3 = pl.pallas_call(
        _smid_body,
        out_shape=[_sds((2 * n, fh3)), _sds((n, f3))])(
            a2, cnt, r2, g2[None, :], be2[None, :], W3lp, W3rp, b3p[None, :])

    a3 = agg3(p3, src, dst)

    out = pl.pallas_call(_sfin_body, out_shape=_sds((n, f3)))(a3, cnt, r3)
    return out[:, :d_out]


# speedup vs baseline: 1.9710x; 1.9710x over previous
import jax
import jax.numpy as jnp
from jax import lax
from jax.experimental import pallas as pl
from jax.experimental.pallas import tpu as pltpu
from jax.experimental.pallas import tpu_sc as plsc

_EPS = 1e-5
_NCORES = 2
_NSUB = 16
_K = 80


def _make_agg(n, e, fh, with_counts):
    ncs = e // (_K * _NSUB)
    nquad = ncs // 4
    na = n + 8
    nr = (na // _NSUB) // 8 * 8
    rem = na - nr * _NSUB
    mesh = plsc.VectorSubcoreMesh(core_axis_name="c", subcore_axis_name="s")

    nbuf = 4
    cw = 16
    out_type = [jax.ShapeDtypeStruct((_NCORES, na, fh), jnp.float32)]
    scratch = [
        pltpu.VMEM((ncs, _K), jnp.int32),
        pltpu.VMEM((ncs, _K), jnp.int32),
        pltpu.VMEM_SHARED((na, fh), jnp.float32),
    ] + [pltpu.VMEM((_K, fh), jnp.float32) for _ in range(nbuf)] \
      + [pltpu.SemaphoreType.DMA for _ in range(2 * nbuf)]
    if with_counts:
        out_type.append(jax.ShapeDtypeStruct((na, cw), jnp.float32))
        scratch += [
            pltpu.VMEM((_K, cw), jnp.float32),
            pltpu.VMEM((_K, cw), jnp.float32),
            pltpu.VMEM_SHARED((na, cw), jnp.float32),
            pltpu.SemaphoreType.DMA,
        ]

    def body(p_hbm, src_hbm, dst_hbm, *rest):
        if with_counts:
            (ones_hbm, zer_hbm, out_hbm, cnt_hbm, srcb, dstb, acc_sh,
             *tl) = rest
            rows = tl[:nbuf]
            gsem = tl[nbuf:2 * nbuf]
            ssem = tl[2 * nbuf:3 * nbuf]
            ones, zb, cnt_sh, csem = tl[3 * nbuf:]
        else:
            (out_hbm, srcb, dstb, acc_sh, *tl) = rest
            rows = tl[:nbuf]
            gsem = tl[nbuf:2 * nbuf]
            ssem = tl[2 * nbuf:3 * nbuf]
        c = lax.axis_index("c")
        s = lax.axis_index("s")

        zeros16 = jnp.zeros((16,), jnp.float32)

        def zrow(r, _):
            for j in range(fh // 16):
                rows[0][r, pl.ds(j * 16, 16)] = zeros16
            return 0

        lax.fori_loop(0, _K, zrow, 0)

        base = s * nr
        full, tail = nr // _K, nr % _K

        def zfill(dst_sh, zsrc):
            for j in range(full):
                pltpu.sync_copy(zsrc.at[pl.ds(0, _K)],
                                dst_sh.at[pl.ds(base + j * _K, _K)])
            if tail:
                pltpu.sync_copy(zsrc.at[pl.ds(0, tail)],
                                dst_sh.at[pl.ds(base + full * _K, tail)])
            if rem:
                @pl.when(s == _NSUB - 1)
                def _():
                    pltpu.sync_copy(zsrc.at[pl.ds(0, rem)],
                                    dst_sh.at[pl.ds(nr * _NSUB, rem)])

        zfill(acc_sh, rows[0])

        if with_counts:
            pltpu.sync_copy(ones_hbm, ones)
            pltpu.sync_copy(zer_hbm, zb)

            @pl.when(c == 0)
            def _():
                zfill(cnt_sh, zb)

        plsc.subcore_barrier()

        row_off = c * n
        pltpu.sync_copy(src_hbm.at[pl.ds(s * ncs, ncs)], srcb)
        pltpu.sync_copy(dst_hbm.at[pl.ds(s * ncs, ncs)], dstb)

        def fixrow(r, _):
            for j in range(_K // 16):
                sl = pl.ds(j * 16, 16)
                srcb[r, sl] = srcb[r, sl] + row_off
            return 0

        lax.fori_loop(0, ncs, fixrow, 0)

        def gstart(j, b):
            pltpu.async_copy(p_hbm.at[srcb.at[j]], rows[b], gsem[b])

        def gwait(j, b):
            pltpu.make_async_copy(p_hbm.at[srcb.at[j]], rows[b],
                                  gsem[b]).wait()

        def sstart(j, b):
            pltpu.async_copy(rows[b], acc_sh.at[dstb.at[j]], ssem[b],
                             add=True)
            if with_counts:
                @pl.when(c == 0)
                def _():
                    pltpu.async_copy(ones, cnt_sh.at[dstb.at[j]], csem,
                                     add=True)

        def swait(j, b):
            pltpu.make_async_copy(rows[b], acc_sh.at[dstb.at[j]],
                                  ssem[b]).wait()

        for b in range(nbuf):
            gstart(b, b)

        def quad(t, _):
            j0 = nbuf * t
            for b in range(nbuf):
                gwait(j0 + b, b)
                sstart(j0 + b, b)
            for b in range(nbuf):
                jn = j0 + b + nbuf

                @pl.when(jn < ncs)
                def _(b=b, j=j0 + b, jn=jn):
                    swait(j, b)
                    gstart(jn, b)

            return 0

        lax.fori_loop(0, nquad, quad, 0)

        for j in range(nquad * nbuf, ncs):
            gwait(j, j % nbuf)
            sstart(j, j % nbuf)
        for j in range(ncs - nbuf, ncs):
            swait(j, j % nbuf)
        if with_counts:
            @pl.when(c == 0)
            def _():
                def cdrain(j, _):
                    pltpu.make_async_copy(ones, cnt_sh.at[dstb.at[0]],
                                          csem).wait()
                    return 0

                lax.fori_loop(0, ncs, cdrain, 0)

        plsc.subcore_barrier()

        def wb(src_sh, dst_hbm_full, lead):
            dst3 = dst_hbm_full.at[lead] if lead is not None else dst_hbm_full
            pltpu.sync_copy(src_sh.at[pl.ds(base, nr)],
                            dst3.at[pl.ds(base, nr)])
            if rem:
                @pl.when(s == _NSUB - 1)
                def _():
                    pltpu.sync_copy(src_sh.at[pl.ds(nr * _NSUB, rem)],
                                    dst3.at[pl.ds(nr * _NSUB, rem)])

        wb(acc_sh, out_hbm, c)
        if with_counts:
            @pl.when(c == 0)
            def _():
                wb(cnt_sh, cnt_hbm, None)

    k = pl.kernel(body, out_type=out_type, mesh=mesh, scratch_types=scratch,
                  compiler_params=pltpu.CompilerParams(
                      use_tc_tiling_on_sc=False))
    if with_counts:
        return k
    return lambda *a: k(*a)[0]


_DN = (((1,), (1,)), ((), ()))


def _split_store(pp, p_ref):
    n, f2 = pp.shape
    fh = f2 // 2
    p_ref[:n, :] = pp[:, :fh]
    p_ref[n:, :] = pp[:, fh:]


def _s0_body(x_ref, wl_ref, wr_ref, b_ref, p_ref, r_ref):
    x = x_ref[...]
    pp = lax.dot_general(x, wl_ref[...], _DN,
                         preferred_element_type=jnp.float32)
    _split_store(pp, p_ref)
    r_ref[...] = lax.dot_general(x, wr_ref[...], _DN,
                                 preferred_element_type=jnp.float32) + b_ref[...]


def _smid_body(a_ref, c_ref, r_ref, g_ref, be_ref, wl_ref, wr_ref, b_ref,
               p_ref, rn_ref):
    n = r_ref.shape[0]
    cnt = jnp.maximum(c_ref[:n, :1], 1.0)
    a = a_ref[...]
    agg = jnp.concatenate([a[0, :n], a[1, :n]], axis=1)
    z = agg / cnt + r_ref[...]
    mu = jnp.mean(z, axis=0, keepdims=True)
    zc = z - mu
    var = jnp.mean(zc * zc, axis=0, keepdims=True)
    h = zc * lax.rsqrt(var + _EPS) * g_ref[...] + be_ref[...]
    h = jnp.maximum(h, 0.0)
    pp = lax.dot_general(h, wl_ref[...], _DN,
                         preferred_element_type=jnp.float32)
    _split_store(pp, p_ref)
    rn_ref[...] = lax.dot_general(h, wr_ref[...], _DN,
                                  preferred_element_type=jnp.float32) + b_ref[...]


def _sfin_body(a_ref, c_ref, r_ref, o_ref):
    n = r_ref.shape[0]
    cnt = jnp.maximum(c_ref[:n, :1], 1.0)
    a = a_ref[...]
    agg = jnp.concatenate([a[0, :n], a[1, :n]], axis=1)
    o_ref[...] = agg / cnt + r_ref[...]


def _sds(shape):
    return jax.ShapeDtypeStruct(shape, jnp.float32)


def kernel(x, edge_index, W1l, b1, W1r, g1, be1, W2l, b2, W2r, g2, be2,
           W3l, b3, W3r):
    n, d_in = x.shape
    e = edge_index.shape[1]
    d_hid = W1l.shape[0]
    d_out = W3l.shape[0]
    f3 = 64
    fh = d_hid // 2
    fh3 = f3 // 2

    grp = _K * _NSUB
    e_pad = -(-e // grp) * grp
    src = jnp.pad(edge_index[0], (0, e_pad - e)).reshape(e_pad // _K, _K)
    dst = jnp.pad(edge_index[1], (0, e_pad - e),
                  constant_values=n).reshape(e_pad // _K, _K)
    ones_c = jnp.ones((_K, 16), jnp.float32)
    zer_c = jnp.zeros((_K, 16), jnp.float32)

    pad = f3 - d_out
    W3lp = jnp.pad(W3l, ((0, pad), (0, 0)))
    W3rp = jnp.pad(W3r, ((0, pad), (0, 0)))
    b3p = jnp.pad(b3, (0, pad))

    agg1 = _make_agg(n, e_pad, fh, True)
    agg2 = _make_agg(n, e_pad, fh, False)
    agg3 = _make_agg(n, e_pad, fh3, False)

    p1, r1 = pl.pallas_call(
        _s0_body,
        out_shape=[_sds((2 * n, fh)), _sds((n, d_hid))])(
            x, W1l, W1r, b1[None, :])

    a1, cnt = agg1(p1, src, dst, ones_c, zer_c)

    p2, r2 = pl.pallas_call(
        _smid_body,
        out_shape=[_sds((2 * n, fh)), _sds((n, d_hid))])(
            a1, cnt, r1, g1[None, :], be1[None, :], W2l, W2r, b2[None, :])

    a2 = agg2(p2, src, dst)

    p3, r3 = pl.pallas_call(
        _smid_body,
        out_shape=[_sds((2 * n, fh3)), _sds((n, f3))])(
            a2, cnt, r2, g2[None, :], be2[None, :], W3lp, W3rp, b3p[None, :])

    a3 = agg3(p3, src, dst)

    out = pl.pallas_call(_sfin_body, out_shape=_sds((n, f3)))(a3, cnt, r3)
    return out[:, :d_out]


# speedup vs baseline: 2.0181x; 1.0239x over previous
import jax
import jax.numpy as jnp
from jax import lax
from jax.experimental import pallas as pl
from jax.experimental.pallas import tpu as pltpu
from jax.experimental.pallas import tpu_sc as plsc

_EPS = 1e-5
_NCORES = 2
_NSUB = 16
_K = 80


def _make_agg(n, e, fh, with_counts):
    ncs = e // (_K * _NSUB)
    nquad = ncs // 4
    na = n + 8
    nr = (na // _NSUB) // 8 * 8
    rem = na - nr * _NSUB
    mesh = plsc.VectorSubcoreMesh(core_axis_name="c", subcore_axis_name="s")

    nbuf = 4
    cw = 16
    out_type = [jax.ShapeDtypeStruct((_NCORES, na, fh), jnp.float32)]
    scratch = [
        pltpu.VMEM((ncs, _K), jnp.int32),
        pltpu.VMEM((ncs, _K), jnp.int32),
        pltpu.VMEM_SHARED((na, fh), jnp.float32),
    ] + [pltpu.VMEM((_K, fh), jnp.float32) for _ in range(nbuf)] \
      + [pltpu.SemaphoreType.DMA for _ in range(2 * nbuf)]
    if with_counts:
        out_type.append(jax.ShapeDtypeStruct((na, cw), jnp.float32))
        scratch += [
            pltpu.VMEM((_K, cw), jnp.float32),
            pltpu.VMEM((_K, cw), jnp.float32),
            pltpu.VMEM_SHARED((na, cw), jnp.float32),
            pltpu.SemaphoreType.DMA,
        ]

    def body(p_hbm, edge_hbm, *rest):
        if with_counts:
            (ones_hbm, zer_hbm, out_hbm, cnt_hbm, srcb, dstb, acc_sh,
             *tl) = rest
            rows = tl[:nbuf]
            gsem = tl[nbuf:2 * nbuf]
            ssem = tl[2 * nbuf:3 * nbuf]
            ones, zb, cnt_sh, csem = tl[3 * nbuf:]
        else:
            (out_hbm, srcb, dstb, acc_sh, *tl) = rest
            rows = tl[:nbuf]
            gsem = tl[nbuf:2 * nbuf]
            ssem = tl[2 * nbuf:3 * nbuf]
        c = lax.axis_index("c")
        s = lax.axis_index("s")

        zeros16 = jnp.zeros((16,), jnp.float32)

        def zrow(r, _):
            for j in range(fh // 16):
                rows[0][r, pl.ds(j * 16, 16)] = zeros16
            return 0

        lax.fori_loop(0, _K, zrow, 0)

        base = s * nr
        full, tail = nr // _K, nr % _K

        def zfill(dst_sh, zsrc):
            for j in range(full):
                pltpu.sync_copy(zsrc.at[pl.ds(0, _K)],
                                dst_sh.at[pl.ds(base + j * _K, _K)])
            if tail:
                pltpu.sync_copy(zsrc.at[pl.ds(0, tail)],
                                dst_sh.at[pl.ds(base + full * _K, tail)])
            if rem:
                @pl.when(s == _NSUB - 1)
                def _():
                    pltpu.sync_copy(zsrc.at[pl.ds(0, rem)],
                                    dst_sh.at[pl.ds(nr * _NSUB, rem)])

        zfill(acc_sh, rows[0])

        if with_counts:
            pltpu.sync_copy(ones_hbm, ones)
            pltpu.sync_copy(zer_hbm, zb)

            @pl.when(c == 0)
            def _():
                zfill(cnt_sh, zb)

        plsc.subcore_barrier()

        row_off = c * n
        pltpu.sync_copy(edge_hbm.at[0, pl.ds(s * ncs, ncs)], srcb)
        pltpu.sync_copy(edge_hbm.at[1, pl.ds(s * ncs, ncs)], dstb)

        def fixrow(r, _):
            for j in range(_K // 16):
                sl = pl.ds(j * 16, 16)
                srcb[r, sl] = srcb[r, sl] + row_off
            return 0

        lax.fori_loop(0, ncs, fixrow, 0)

        def gstart(j, b):
            pltpu.async_copy(p_hbm.at[srcb.at[j]], rows[b], gsem[b])

        def gwait(j, b):
            pltpu.make_async_copy(p_hbm.at[srcb.at[j]], rows[b],
                                  gsem[b]).wait()

        def sstart(j, b):
            pltpu.async_copy(rows[b], acc_sh.at[dstb.at[j]], ssem[b],
                             add=True)
            if with_counts:
                @pl.when(c == 0)
                def _():
                    pltpu.async_copy(ones, cnt_sh.at[dstb.at[j]], csem,
                                     add=True)

        def swait(j, b):
            pltpu.make_async_copy(rows[b], acc_sh.at[dstb.at[j]],
                                  ssem[b]).wait()

        for b in range(nbuf):
            gstart(b, b)

        def quad(t, _):
            j0 = nbuf * t
            for b in range(nbuf):
                gwait(j0 + b, b)
                sstart(j0 + b, b)
            for b in range(nbuf):
                jn = j0 + b + nbuf

                @pl.when(jn < ncs)
                def _(b=b, j=j0 + b, jn=jn):
                    swait(j, b)
                    gstart(jn, b)

            return 0

        lax.fori_loop(0, nquad, quad, 0)

        for j in range(nquad * nbuf, ncs):
            gwait(j, j % nbuf)
            sstart(j, j % nbuf)
        for j in range(ncs - nbuf, ncs):
            swait(j, j % nbuf)
        if with_counts:
            @pl.when(c == 0)
            def _():
                def cdrain(j, _):
                    pltpu.make_async_copy(ones, cnt_sh.at[dstb.at[0]],
                                          csem).wait()
                    return 0

                lax.fori_loop(0, ncs, cdrain, 0)

        plsc.subcore_barrier()

        def wb(src_sh, dst_hbm_full, lead):
            dst3 = dst_hbm_full.at[lead] if lead is not None else dst_hbm_full
            pltpu.sync_copy(src_sh.at[pl.ds(base, nr)],
                            dst3.at[pl.ds(base, nr)])
            if rem:
                @pl.when(s == _NSUB - 1)
                def _():
                    pltpu.sync_copy(src_sh.at[pl.ds(nr * _NSUB, rem)],
                                    dst3.at[pl.ds(nr * _NSUB, rem)])

        wb(acc_sh, out_hbm, c)
        if with_counts:
            @pl.when(c == 0)
            def _():
                wb(cnt_sh, cnt_hbm, None)

    k = pl.kernel(body, out_type=out_type, mesh=mesh, scratch_types=scratch,
                  compiler_params=pltpu.CompilerParams(
                      use_tc_tiling_on_sc=False))
    if with_counts:
        return k
    return lambda *a: k(*a)[0]


_DN = (((1,), (1,)), ((), ()))


def _split_store(pp, p_ref):
    n, f2 = pp.shape
    fh = f2 // 2
    p_ref[:n, :] = pp[:, :fh]
    p_ref[n:, :] = pp[:, fh:]


def _s0_body(x_ref, wl_ref, wr_ref, b_ref, p_ref, r_ref):
    x = x_ref[...]
    pp = lax.dot_general(x, wl_ref[...], _DN,
                         preferred_element_type=jnp.float32)
    _split_store(pp, p_ref)
    r_ref[...] = lax.dot_general(x, wr_ref[...], _DN,
                                 preferred_element_type=jnp.float32) + b_ref[...]


def _smid_body(a_ref, c_ref, r_ref, g_ref, be_ref, wl_ref, wr_ref, b_ref,
               p_ref, rn_ref):
    n = r_ref.shape[0]
    cnt = jnp.maximum(c_ref[:n, :1], 1.0)
    a = a_ref[...]
    agg = jnp.concatenate([a[0, :n], a[1, :n]], axis=1)
    z = agg / cnt + r_ref[...]
    mu = jnp.mean(z, axis=0, keepdims=True)
    zc = z - mu
    var = jnp.mean(zc * zc, axis=0, keepdims=True)
    h = zc * lax.rsqrt(var + _EPS) * g_ref[...] + be_ref[...]
    h = jnp.maximum(h, 0.0)
    pp = lax.dot_general(h, wl_ref[...], _DN,
                         preferred_element_type=jnp.float32)
    _split_store(pp, p_ref)
    rn_ref[...] = lax.dot_general(h, wr_ref[...], _DN,
                                  preferred_element_type=jnp.float32) + b_ref[...]


def _s2_body(a_ref, c_ref, r_ref, g_ref, be_ref, wl_ref, wr_ref, b_ref,
             p_ref, rn_ref):
    n = r_ref.shape[0]
    fpad = 2 * p_ref.shape[1] - wl_ref.shape[0]
    cnt = jnp.maximum(c_ref[:n, :1], 1.0)
    a = a_ref[...]
    agg = jnp.concatenate([a[0, :n], a[1, :n]], axis=1)
    z = agg / cnt + r_ref[...]
    mu = jnp.mean(z, axis=0, keepdims=True)
    zc = z - mu
    var = jnp.mean(zc * zc, axis=0, keepdims=True)
    h = zc * lax.rsqrt(var + _EPS) * g_ref[...] + be_ref[...]
    h = jnp.maximum(h, 0.0)
    pp = lax.dot_general(h, wl_ref[...], _DN,
                         preferred_element_type=jnp.float32)
    pp = jnp.concatenate([pp, jnp.zeros((n, fpad), jnp.float32)], axis=1)
    _split_store(pp, p_ref)
    rn_ref[...] = lax.dot_general(h, wr_ref[...], _DN,
                                  preferred_element_type=jnp.float32) + b_ref[...]


def _sfin_body(a_ref, c_ref, r_ref, o_ref):
    n, d_out = r_ref.shape
    cnt = jnp.maximum(c_ref[:n, :1], 1.0)
    a = a_ref[...]
    agg = jnp.concatenate([a[0, :n], a[1, :n]], axis=1)
    o_ref[...] = agg[:, :d_out] / cnt + r_ref[...]


def _sds(shape):
    return jax.ShapeDtypeStruct(shape, jnp.float32)


def kernel(x, edge_index, W1l, b1, W1r, g1, be1, W2l, b2, W2r, g2, be2,
           W3l, b3, W3r):
    n, d_in = x.shape
    e = edge_index.shape[1]
    d_hid = W1l.shape[0]
    d_out = W3l.shape[0]
    f3 = 64
    fh = d_hid // 2
    fh3 = f3 // 2

    edges = edge_index.reshape(2, e // _K, _K)
    ones_c = jnp.ones((_K, 16), jnp.float32)
    zer_c = jnp.zeros((_K, 16), jnp.float32)

    agg1 = _make_agg(n, e, fh, True)
    agg2 = _make_agg(n, e, fh, False)
    agg3 = _make_agg(n, e, fh3, False)

    p1, r1 = pl.pallas_call(
        _s0_body,
        out_shape=[_sds((2 * n, fh)), _sds((n, d_hid))])(
            x, W1l, W1r, b1[None, :])

    a1, cnt = agg1(p1, edges, ones_c, zer_c)

    p2, r2 = pl.pallas_call(
        _smid_body,
        out_shape=[_sds((2 * n, fh)), _sds((n, d_hid))])(
            a1, cnt, r1, g1[None, :], be1[None, :], W2l, W2r, b2[None, :])

    a2 = agg2(p2, edges)

    p3, r3 = pl.pallas_call(
        _s2_body,
        out_shape=[_sds((2 * n, fh3)), _sds((n, d_out))])(
            a2, cnt, r2, g2[None, :], be2[None, :], W3l, W3r, b3[None, :])

    a3 = agg3(p3, edges)

    return pl.pallas_call(_sfin_body, out_shape=_sds((n, d_out)))(a3, cnt, r3)


# speedup vs baseline: 2.0422x; 1.0120x over previous
import jax
import jax.numpy as jnp
from jax import lax
from jax.experimental import pallas as pl
from jax.experimental.pallas import tpu as pltpu
from jax.experimental.pallas import tpu_sc as plsc

_EPS = 1e-5
_NCORES = 2
_NSUB = 16
_K = 80


def _make_agg(n, e, fh, with_counts):
    ncs = e // (_K * _NSUB)
    nquad = ncs // 4
    na = n + 8
    nr = (na // _NSUB) // 8 * 8
    rem = na - nr * _NSUB
    mesh = plsc.VectorSubcoreMesh(core_axis_name="c", subcore_axis_name="s")

    nbuf = 4
    cw = 16
    out_type = [jax.ShapeDtypeStruct((_NCORES, na, fh), jnp.float32)]
    scratch = [
        pltpu.VMEM((ncs, _K), jnp.int32),
        pltpu.VMEM((ncs, _K), jnp.int32),
        pltpu.VMEM_SHARED((na, fh), jnp.float32),
    ] + [pltpu.VMEM((_K, fh), jnp.float32) for _ in range(nbuf)] \
      + [pltpu.SemaphoreType.DMA for _ in range(2 * nbuf + 1)]
    if with_counts:
        out_type.append(jax.ShapeDtypeStruct((na, cw), jnp.float32))
        scratch += [
            pltpu.VMEM((_K, cw), jnp.float32),
            pltpu.VMEM((_K, cw), jnp.float32),
            pltpu.VMEM_SHARED((na, cw), jnp.float32),
            pltpu.SemaphoreType.DMA,
        ]

    def body(p_hbm, edge_hbm, *rest):
        if with_counts:
            (ones_hbm, zer_hbm, out_hbm, cnt_hbm, srcb, dstb, acc_sh,
             *tl) = rest
            rows = tl[:nbuf]
            gsem = tl[nbuf:2 * nbuf]
            ssem = tl[2 * nbuf:3 * nbuf]
            zsem = tl[3 * nbuf]
            ones, zb, cnt_sh, csem = tl[3 * nbuf + 1:]
        else:
            (out_hbm, srcb, dstb, acc_sh, *tl) = rest
            rows = tl[:nbuf]
            gsem = tl[nbuf:2 * nbuf]
            ssem = tl[2 * nbuf:3 * nbuf]
            zsem = tl[3 * nbuf]
        c = lax.axis_index("c")
        s = lax.axis_index("s")

        zeros16 = jnp.zeros((16,), jnp.float32)

        def zrow(r, _):
            for j in range(fh // 16):
                rows[0][r, pl.ds(j * 16, 16)] = zeros16
            return 0

        lax.fori_loop(0, _K, zrow, 0)

        base = s * nr
        full, tail = nr // _K, nr % _K

        def zfill(dst_sh, zsrc, fire):
            def op(src_sl, dst_sl):
                if fire:
                    pltpu.async_copy(src_sl, dst_sl, zsem)
                else:
                    pltpu.make_async_copy(src_sl, dst_sl, zsem).wait()

            for j in range(full):
                op(zsrc.at[pl.ds(0, _K)], dst_sh.at[pl.ds(base + j * _K, _K)])
            if tail:
                op(zsrc.at[pl.ds(0, tail)],
                   dst_sh.at[pl.ds(base + full * _K, tail)])
            if rem:
                @pl.when(s == _NSUB - 1)
                def _():
                    op(zsrc.at[pl.ds(0, rem)],
                       dst_sh.at[pl.ds(nr * _NSUB, rem)])

        def zfill_all(fire):
            zfill(acc_sh, rows[0], fire)
            if with_counts:
                @pl.when(c == 0)
                def _():
                    zfill(cnt_sh, zb, fire)

        if with_counts:
            pltpu.sync_copy(ones_hbm, ones)
            pltpu.sync_copy(zer_hbm, zb)

        zfill_all(True)

        row_off = c * n
        pltpu.sync_copy(edge_hbm.at[0, pl.ds(s * ncs, ncs)], srcb)
        pltpu.sync_copy(edge_hbm.at[1, pl.ds(s * ncs, ncs)], dstb)

        def fixrow(r, _):
            for j in range(_K // 16):
                sl = pl.ds(j * 16, 16)
                srcb[r, sl] = srcb[r, sl] + row_off
            return 0

        lax.fori_loop(0, ncs, fixrow, 0)
        zfill_all(False)

        def gstart(j, b):
            pltpu.async_copy(p_hbm.at[srcb.at[j]], rows[b], gsem[b])

        def gwait(j, b):
            pltpu.make_async_copy(p_hbm.at[srcb.at[j]], rows[b],
                                  gsem[b]).wait()

        def sstart(j, b):
            pltpu.async_copy(rows[b], acc_sh.at[dstb.at[j]], ssem[b],
                             add=True)
            if with_counts:
                @pl.when(c == 0)
                def _():
                    pltpu.async_copy(ones, cnt_sh.at[dstb.at[j]], csem,
                                     add=True)

        def swait(j, b):
            pltpu.make_async_copy(rows[b], acc_sh.at[dstb.at[j]],
                                  ssem[b]).wait()

        for b in range(nbuf):
            gstart(b, b)

        plsc.subcore_barrier()

        def quad(t, _):
            j0 = nbuf * t
            for b in range(nbuf):
                gwait(j0 + b, b)
                sstart(j0 + b, b)
            for b in range(nbuf):
                jn = j0 + b + nbuf

                @pl.when(jn < ncs)
                def _(b=b, j=j0 + b, jn=jn):
                    swait(j, b)
                    gstart(jn, b)

            return 0

        lax.fori_loop(0, nquad, quad, 0)

        for j in range(nquad * nbuf, ncs):
            gwait(j, j % nbuf)
            sstart(j, j % nbuf)
        for j in range(ncs - nbuf, ncs):
            swait(j, j % nbuf)
        if with_counts:
            @pl.when(c == 0)
            def _():
                def cdrain(j, _):
                    pltpu.make_async_copy(ones, cnt_sh.at[dstb.at[0]],
                                          csem).wait()
                    return 0

                lax.fori_loop(0, ncs, cdrain, 0)

        plsc.subcore_barrier()

        def wb(src_sh, dst_hbm_full, lead):
            dst3 = dst_hbm_full.at[lead] if lead is not None else dst_hbm_full
            pltpu.sync_copy(src_sh.at[pl.ds(base, nr)],
                            dst3.at[pl.ds(base, nr)])
            if rem:
                @pl.when(s == _NSUB - 1)
                def _():
                    pltpu.sync_copy(src_sh.at[pl.ds(nr * _NSUB, rem)],
                                    dst3.at[pl.ds(nr * _NSUB, rem)])

        wb(acc_sh, out_hbm, c)
        if with_counts:
            @pl.when(c == 0)
            def _():
                wb(cnt_sh, cnt_hbm, None)

    k = pl.kernel(body, out_type=out_type, mesh=mesh, scratch_types=scratch,
                  compiler_params=pltpu.CompilerParams(
                      use_tc_tiling_on_sc=False))
    if with_counts:
        return k
    return lambda *a: k(*a)[0]


_DN = (((1,), (1,)), ((), ()))


def _split_store(pp, p_ref):
    n, f2 = pp.shape
    fh = f2 // 2
    p_ref[:n, :] = pp[:, :fh]
    p_ref[n:, :] = pp[:, fh:]


def _s0_body(x_ref, wl_ref, wr_ref, b_ref, p_ref, r_ref):
    x = x_ref[...]
    pp = lax.dot_general(x, wl_ref[...], _DN,
                         preferred_element_type=jnp.float32)
    _split_store(pp, p_ref)
    r_ref[...] = lax.dot_general(x, wr_ref[...], _DN,
                                 preferred_element_type=jnp.float32) + b_ref[...]


def _smid_body(a_ref, c_ref, r_ref, g_ref, be_ref, wl_ref, wr_ref, b_ref,
               p_ref, rn_ref):
    n = r_ref.shape[0]
    cnt = jnp.maximum(c_ref[:n, :1], 1.0)
    a = a_ref[...]
    agg = jnp.concatenate([a[0, :n], a[1, :n]], axis=1)
    z = agg / cnt + r_ref[...]
    mu = jnp.mean(z, axis=0, keepdims=True)
    zc = z - mu
    var = jnp.mean(zc * zc, axis=0, keepdims=True)
    h = zc * lax.rsqrt(var + _EPS) * g_ref[...] + be_ref[...]
    h = jnp.maximum(h, 0.0)
    pp = lax.dot_general(h, wl_ref[...], _DN,
                         preferred_element_type=jnp.float32)
    _split_store(pp, p_ref)
    rn_ref[...] = lax.dot_general(h, wr_ref[...], _DN,
                                  preferred_element_type=jnp.float32) + b_ref[...]


def _s2_body(a_ref, c_ref, r_ref, g_ref, be_ref, wl_ref, wr_ref, b_ref,
             p_ref, rn_ref):
    n = r_ref.shape[0]
    fpad = 2 * p_ref.shape[1] - wl_ref.shape[0]
    cnt = jnp.maximum(c_ref[:n, :1], 1.0)
    a = a_ref[...]
    agg = jnp.concatenate([a[0, :n], a[1, :n]], axis=1)
    z = agg / cnt + r_ref[...]
    mu = jnp.mean(z, axis=0, keepdims=True)
    zc = z - mu
    var = jnp.mean(zc * zc, axis=0, keepdims=True)
    h = zc * lax.rsqrt(var + _EPS) * g_ref[...] + be_ref[...]
    h = jnp.maximum(h, 0.0)
    pp = lax.dot_general(h, wl_ref[...], _DN,
                         preferred_element_type=jnp.float32)
    pp = jnp.concatenate([pp, jnp.zeros((n, fpad), jnp.float32)], axis=1)
    _split_store(pp, p_ref)
    rn_ref[...] = lax.dot_general(h, wr_ref[...], _DN,
                                  preferred_element_type=jnp.float32) + b_ref[...]


def _sfin_body(a_ref, c_ref, r_ref, o_ref):
    n, d_out = r_ref.shape
    cnt = jnp.maximum(c_ref[:n, :1], 1.0)
    a = a_ref[...]
    agg = jnp.concatenate([a[0, :n], a[1, :n]], axis=1)
    o_ref[...] = agg[:, :d_out] / cnt + r_ref[...]


def _sds(shape):
    return jax.ShapeDtypeStruct(shape, jnp.float32)


def kernel(x, edge_index, W1l, b1, W1r, g1, be1, W2l, b2, W2r, g2, be2,
           W3l, b3, W3r):
    n, d_in = x.shape
    e = edge_index.shape[1]
    d_hid = W1l.shape[0]
    d_out = W3l.shape[0]
    f3 = 64
    fh = d_hid // 2
    fh3 = f3 // 2

    edges = edge_index.reshape(2, e // _K, _K)
    ones_c = jnp.ones((_K, 16), jnp.float32)
    zer_c = jnp.zeros((_K, 16), jnp.float32)

    agg1 = _make_agg(n, e, fh, True)
    agg2 = _make_agg(n, e, fh, False)
    agg3 = _make_agg(n, e, fh3, False)

    p1, r1 = pl.pallas_call(
        _s0_body,
        out_shape=[_sds((2 * n, fh)), _sds((n, d_hid))])(
            x, W1l, W1r, b1[None, :])

    a1, cnt = agg1(p1, edges, ones_c, zer_c)

    p2, r2 = pl.pallas_call(
        _smid_body,
        out_shape=[_sds((2 * n, fh)), _sds((n, d_hid))])(
            a1, cnt, r1, g1[None, :], be1[None, :], W2l, W2r, b2[None, :])

    a2 = agg2(p2, edges)

    p3, r3 = pl.pallas_call(
        _s2_body,
        out_shape=[_sds((2 * n, fh3)), _sds((n, d_out))])(
            a2, cnt, r2, g2[None, :], be2[None, :], W3l, W3r, b3[None, :])

    a3 = agg3(p3, edges)

    return pl.pallas_call(_sfin_body, out_shape=_sds((n, d_out)))(a3, cnt, r3)
